# Initial kernel scaffold; baseline (speedup 1.0000x reference)
import jax
import jax.numpy as jnp
from jax.experimental import pallas as pl


def kernel(x, W, b):
    raise NotImplementedError("write your pallas kernel here")


# speedup vs baseline: 7.2834x; 7.2834x over previous
import functools

import jax
import jax.numpy as jnp
from jax.experimental import pallas as pl

K = 16
ROWS = 256
BIG = 3.0e38


def _edgeconv_tile(x_tile_ref, x_full_ref, w_ref, b_ref, out_ref):
    xt = x_tile_ref[0]
    xf = x_full_ref[0]
    n = xf.shape[0]
    r = xt.shape[0]

    sqf = jnp.sum(xf * xf, axis=1)
    sqt = jnp.sum(xt * xt, axis=1)
    g = jnp.dot(xt, xf.T, preferred_element_type=jnp.float32)
    d = sqt[:, None] + sqf[None, :] - 2.0 * g

    w1 = w_ref[0:32, :]
    wd = w_ref[32:64, :] - w1
    y1 = jnp.dot(xf, w1, preferred_element_type=jnp.float32)
    y2 = jnp.dot(xt, wd, preferred_element_type=jnp.float32) + b_ref[0]

    iota = jax.lax.broadcasted_iota(jnp.int32, (r, n), 1)

    def body(_, carry):
        dcur, acc = carry
        m = jnp.min(dcur, axis=1, keepdims=True)
        t = jnp.where(dcur <= m, iota, n)
        am = jnp.min(t, axis=1, keepdims=True)
        hit = t <= am
        sel = hit.astype(jnp.float32)
        acc = jnp.maximum(acc, jnp.dot(sel, y1, preferred_element_type=jnp.float32))
        dcur = jnp.where(hit, BIG, dcur)
        return dcur, acc

    acc0 = jnp.full((r, 64), -BIG, dtype=jnp.float32)
    _, acc = jax.lax.fori_loop(0, K, body, (d, acc0))
    out_ref[0] = acc + y2


@jax.jit
def kernel(x, W, b):
    B, n, c = x.shape
    b2 = b.reshape(1, 64)
    grid = (B, n // ROWS)
    out = pl.pallas_call(
        _edgeconv_tile,
        grid=grid,
        in_specs=[
            pl.BlockSpec((1, ROWS, c), lambda bb, it: (bb, it, 0)),
            pl.BlockSpec((1, n, c), lambda bb, it: (bb, 0, 0)),
            pl.BlockSpec((64, 64), lambda bb, it: (0, 0)),
            pl.BlockSpec((1, 64), lambda bb, it: (0, 0)),
        ],
        out_specs=pl.BlockSpec((1, ROWS, 64), lambda bb, it: (bb, it, 0)),
        out_shape=jax.ShapeDtypeStruct((B, n, 64), jnp.float32),
    )(x, x, W, b2)
    return out


# speedup vs baseline: 13.5466x; 1.8599x over previous
import functools

import jax
import jax.numpy as jnp
from jax import lax
from jax.experimental import pallas as pl
from jax.experimental.pallas import tpu as pltpu
from jax.experimental.pallas import tpu_sc as plsc

K = 16
ROWS = 256
BIG = 3.0e38
CP = 8


def _knn_tile(x_tile_ref, x_full_ref, w_ref, b_ref, ind_ref, y1_ref, y2_ref):
    bb = pl.program_id(0)
    xt = x_tile_ref[0]
    xf = x_full_ref[0]
    n = xf.shape[0]
    r = xt.shape[0]

    sqf = jnp.sum(xf * xf, axis=1)
    sqt = jnp.sum(xt * xt, axis=1)
    g = jnp.dot(xt, xf.T, preferred_element_type=jnp.float32)
    d = sqt[:, None] + sqf[None, :] - 2.0 * g

    w1 = w_ref[0:32, :]
    wd = w_ref[32:64, :] - w1
    y1_ref[0] = jnp.dot(xt, w1, preferred_element_type=jnp.float32)
    y2_ref[0] = jnp.dot(xt, wd, preferred_element_type=jnp.float32) + b_ref[0]

    iota = lax.broadcasted_iota(jnp.int32, (r, n), 1)
    lane16 = lax.broadcasted_iota(jnp.int32, (r, K), 1)
    gbase = bb * n

    idx_acc = jnp.zeros((r, K), dtype=jnp.int32)
    for t in range(K):
        m = jnp.min(d, axis=1, keepdims=True)
        tt = jnp.where(d <= m, iota, n)
        am = jnp.min(tt, axis=1)
        idx_acc = idx_acc + jnp.where(lane16 == t, (am + gbase)[:, None], 0)
        if t < K - 1:
            d = jnp.where(tt <= am[:, None], BIG, d)
    ind_ref[0] = idx_acc


def _sc_gather_max(nw, ppw, y1_hbm, ind_hbm, y2_hbm, out_hbm,
                   idx_v, rows_v, y2_v, out_v, sem):
    nc = 2
    wid = lax.axis_index("s") * nc + lax.axis_index("c")
    nchunks = ppw // CP

    def chunk_body(c, _):
        pbase = wid * ppw + c * CP
        pltpu.sync_copy(ind_hbm.at[pl.ds(pbase * K, CP * K)], idx_v)
        pltpu.async_copy(y1_hbm.at[idx_v], rows_v, sem).wait()
        pltpu.sync_copy(y2_hbm.at[pl.ds(pbase, CP)], y2_v)

        def point_body(p, _):
            for gq in range(4):
                sl = pl.ds(gq * 16, 16)
                acc = rows_v[p * K, sl]
                for rr in range(1, K):
                    acc = jnp.maximum(acc, rows_v[p * K + rr, sl])
                out_v[p, sl] = acc + y2_v[p, sl]
            return 0

        lax.fori_loop(0, CP, point_body, 0)
        pltpu.sync_copy(out_v, out_hbm.at[pl.ds(pbase, CP)])
        return 0

    lax.fori_loop(0, nchunks, chunk_body, 0)


@jax.jit
def kernel(x, W, b):
    B, n, c = x.shape
    b2 = b.reshape(1, 64)
    grid = (B, n // ROWS)
    ind, y1, y2 = pl.pallas_call(
        _knn_tile,
        grid=grid,
        in_specs=[
            pl.BlockSpec((1, ROWS, c), lambda bb, it: (bb, it, 0)),
            pl.BlockSpec((1, n, c), lambda bb, it: (bb, 0, 0)),
            pl.BlockSpec((64, 64), lambda bb, it: (0, 0)),
            pl.BlockSpec((1, 64), lambda bb, it: (0, 0)),
        ],
        out_specs=[
            pl.BlockSpec((1, ROWS, K), lambda bb, it: (bb, it, 0)),
            pl.BlockSpec((1, ROWS, 64), lambda bb, it: (bb, it, 0)),
            pl.BlockSpec((1, ROWS, 64), lambda bb, it: (bb, it, 0)),
        ],
        out_shape=[
            jax.ShapeDtypeStruct((B, n, K), jnp.int32),
            jax.ShapeDtypeStruct((B, n, 64), jnp.float32),
            jax.ShapeDtypeStruct((B, n, 64), jnp.float32),
        ],
    )(x, x, W, b2)

    npts = B * n
    nw = 32
    ppw = npts // nw
    y1f = y1.reshape(npts, 64)
    y2f = y2.reshape(npts, 64)
    indf = ind.reshape(npts * K)

    mesh = plsc.VectorSubcoreMesh(core_axis_name="c", subcore_axis_name="s")
    sc_fn = pl.kernel(
        functools.partial(_sc_gather_max, nw, ppw),
        out_type=jax.ShapeDtypeStruct((npts, 64), jnp.float32),
        mesh=mesh,
        compiler_params=pltpu.CompilerParams(use_tc_tiling_on_sc=False),
        scratch_types=[
            pltpu.VMEM((CP * K,), jnp.int32),
            pltpu.VMEM((CP * K, 64), jnp.float32),
            pltpu.VMEM((CP, 64), jnp.float32),
            pltpu.VMEM((CP, 64), jnp.float32),
            pltpu.SemaphoreType.DMA,
        ],
    )
    out = sc_fn(y1f, indf, y2f)
    return out.reshape(B, n, 64)
